# baseline (speedup 1.0000x reference)
import functools

import jax
import jax.numpy as jnp
from jax import lax
from jax.experimental import pallas as pl
from jax.experimental.pallas import tpu as pltpu
from jax.experimental.pallas import tpu_sc as plsc

_NC = 2
_NS = 16
_NW = _NC * _NS


def _argmin_body(z_ref, cbt_ref, idx_ref):
    zb = z_ref[...]
    cbt = cbt_ref[...]
    zn = jnp.sum(zb * zb, axis=1, keepdims=True)
    cn = jnp.sum(cbt * cbt, axis=0, keepdims=True)
    d2 = zn - 2.0 * jnp.dot(zb, cbt, preferred_element_type=jnp.float32) + cn
    d2 = jnp.maximum(d2, 0.0)
    m = jnp.min(d2, axis=1, keepdims=True)
    iota = lax.broadcasted_iota(jnp.int32, d2.shape, 1)
    idx = jnp.min(jnp.where(d2 == m, iota, jnp.int32(1 << 30)),
                  axis=1, keepdims=True)
    idx_ref[...] = idx


def _argmin_tc(z2d, cbt, block=256, interpret=False):
    n = z2d.shape[0]
    k = cbt.shape[1]
    grid = n // block
    return pl.pallas_call(
        _argmin_body,
        grid=(grid,),
        in_specs=[
            pl.BlockSpec((block, z2d.shape[1]), lambda i: (i, 0)),
            pl.BlockSpec((z2d.shape[1], k), lambda i: (0, 0)),
        ],
        out_specs=pl.BlockSpec((block, 1), lambda i: (i, 0)),
        out_shape=jax.ShapeDtypeStruct((n, 1), jnp.int32),
        interpret=interpret,
    )(z2d, cbt)


def _gather_sc(table, idx):
    b = idx.shape[0]
    d = table.shape[1]
    bpw = b // _NW
    mesh = plsc.VectorSubcoreMesh(core_axis_name="c", subcore_axis_name="s")

    @functools.partial(
        pl.kernel,
        mesh=mesh,
        out_type=jax.ShapeDtypeStruct((b, d), jnp.float32),
        compiler_params=pltpu.CompilerParams(use_tc_tiling_on_sc=False),
        scratch_types=[
            pltpu.VMEM((bpw,), jnp.int32),
            pltpu.VMEM((bpw, d), jnp.float32),
            pltpu.SemaphoreType.DMA,
        ],
    )
    def gk(table_hbm, idx_hbm, out_hbm, idx_v, rows_v, sem):
        wid = lax.axis_index("s") * _NC + lax.axis_index("c")
        base = wid * bpw
        pltpu.sync_copy(idx_hbm.at[pl.ds(base, bpw)], idx_v)
        pltpu.async_copy(table_hbm.at[idx_v], rows_v, sem).wait()
        pltpu.sync_copy(rows_v, out_hbm.at[pl.ds(base, bpw)])

    return gk(table, idx)


def kernel(z, codebook):
    d = codebook.shape[1]
    z2d = z.reshape(-1, d)
    idx = _argmin_tc(z2d, codebook.T).reshape(-1)
    out = _gather_sc(codebook, idx)
    return out.reshape(z.shape)
